# baseline (speedup 1.0000x reference)
import functools

import jax
import jax.numpy as jnp
from jax import lax
from jax.experimental import pallas as pl
from jax.experimental.pallas import tpu as pltpu
from jax.experimental.pallas import tpu_sc as plsc

B = 16384
EMB_K = 16
NC = 2
NS = 16
NW = NC * NS
BPW = B // NW
CHUNK = 128
NCHUNK = BPW // CHUNK


def _gather_body(uidx_hbm, iidx_hbm, w_hbm, h_hbm, uout_hbm, vout_hbm,
                 uidx_v, iidx_v, urows_v, vrows_v, sem_u, sem_v):
    wid = lax.axis_index("s") * NC + lax.axis_index("c")
    base = wid * BPW
    pltpu.sync_copy(uidx_hbm.at[pl.ds(base, BPW)], uidx_v)
    pltpu.sync_copy(iidx_hbm.at[pl.ds(base, BPW)], iidx_v)
    copies = []
    for j in range(NCHUNK):
        sl = pl.ds(j * CHUNK, CHUNK)
        copies.append(pltpu.async_copy(w_hbm.at[uidx_v.at[sl]], urows_v.at[sl], sem_u))
        copies.append(pltpu.async_copy(h_hbm.at[iidx_v.at[sl]], vrows_v.at[sl], sem_v))
    for c in copies:
        c.wait()
    pltpu.sync_copy(urows_v, uout_hbm.at[pl.ds(base, BPW)])
    pltpu.sync_copy(vrows_v, vout_hbm.at[pl.ds(base, BPW)])


_gather = functools.partial(
    pl.kernel,
    mesh=plsc.VectorSubcoreMesh(core_axis_name="c", subcore_axis_name="s"),
    compiler_params=pltpu.CompilerParams(use_tc_tiling_on_sc=False),
    out_type=[
        jax.ShapeDtypeStruct((B, EMB_K), jnp.float32),
        jax.ShapeDtypeStruct((B, EMB_K), jnp.float32),
    ],
    scratch_types=[
        pltpu.VMEM((BPW,), jnp.int32),
        pltpu.VMEM((BPW,), jnp.int32),
        pltpu.VMEM((BPW, EMB_K), jnp.float32),
        pltpu.VMEM((BPW, EMB_K), jnp.float32),
        pltpu.SemaphoreType.DMA,
        pltpu.SemaphoreType.DMA,
    ],
)(_gather_body)


def _mlp_body(u_ref, v_ref, w1u_ref, w1v_ref, b1_ref, w2t_ref, o_ref):
    h = (
        jnp.dot(u_ref[...], w1u_ref[...], preferred_element_type=jnp.float32)
        + jnp.dot(v_ref[...], w1v_ref[...], preferred_element_type=jnp.float32)
        + b1_ref[...]
    )
    h = jnp.maximum(h, 0.0)
    o_ref[...] = jnp.dot(h, w2t_ref[...], preferred_element_type=jnp.float32)


def _mlp(u, v, w1u, w1v, b1_2d, w2t):
    return pl.pallas_call(
        _mlp_body,
        out_shape=jax.ShapeDtypeStruct((B, 1), jnp.float32),
    )(u, v, w1u, w1v, b1_2d, w2t)


@jax.jit
def kernel(x, W, H, W1, b1, W2):
    user_idx = x[:, 0]
    item_idx = x[:, 1]
    U, V = _gather(user_idx, item_idx, W, H)
    w1u = W1[:, :EMB_K].T
    w1v = W1[:, EMB_K:].T
    return _mlp(U, V, w1u, w1v, b1.reshape(1, EMB_K), W2.T)
